# Initial kernel scaffold; baseline (speedup 1.0000x reference)
import jax
import jax.numpy as jnp
from jax.experimental import pallas as pl


def kernel(global_features, W1, b1, W2, b2, ema_load):
    raise NotImplementedError("write your pallas kernel here")


# speedup vs baseline: 5.3640x; 5.3640x over previous
import functools

import jax
import jax.numpy as jnp
from jax.experimental import pallas as pl
from jax.experimental.pallas import tpu as pltpu


_K = 8


def _router_body(x_ref, w1_ref, b1_ref, w2_ref, b2_ref, ema_ref,
                 coeffs_ref, mon_ref, cv_ref, *, n_blocks, n_rows, n_experts):
    i = pl.program_id(0)

    z = jax.lax.dot_general(x_ref[...], w1_ref[...],
                            (((1,), (1,)), ((), ())),
                            preferred_element_type=jnp.float32)
    h = jax.nn.silu(z + b1_ref[...])
    logits = jax.lax.dot_general(h, w2_ref[...],
                                 (((1,), (1,)), ((), ())),
                                 preferred_element_type=jnp.float32)
    scores = jax.nn.sigmoid(logits + b2_ref[...])

    iota = jax.lax.broadcasted_iota(jnp.int32, scores.shape, 1)
    masked = scores
    sel = jnp.zeros(scores.shape, jnp.bool_)
    for _ in range(_K):
        m = jnp.max(masked, axis=1, keepdims=True)
        elig = masked == m
        fidx = jnp.min(jnp.where(elig, iota, n_experts), axis=1, keepdims=True)
        first = iota == fidx
        sel = jnp.logical_or(sel, first)
        masked = jnp.where(first, -1.0, masked)

    selscores = jnp.where(sel, scores, 0.0)
    denom = jnp.sum(selscores, axis=1, keepdims=True) + 1e-8
    coeffs_ref[...] = selscores / denom

    rowmax = jnp.max(scores, axis=1, keepdims=True)
    part = jnp.sum(rowmax / denom)

    @pl.when(i == 0)
    def _init():
        mon_ref[0, 0] = 0.0
        e = ema_ref[...]
        mu = jnp.sum(e) / n_experts
        var = jnp.sum((e - mu) ** 2) / (n_experts - 1)
        cv_ref[0, 0] = jnp.sqrt(var) / (mu + 1e-8)

    mon_ref[0, 0] = mon_ref[0, 0] + part

    @pl.when(i == n_blocks - 1)
    def _final():
        mon_ref[0, 0] = mon_ref[0, 0] / n_rows


def kernel(global_features, W1, b1, W2, b2, ema_load):
    n, d = global_features.shape
    h_dim = W1.shape[0]
    e_dim = W2.shape[0]
    bn = 512
    n_blocks = n // bn

    body = functools.partial(_router_body, n_blocks=n_blocks, n_rows=n,
                             n_experts=e_dim)
    coeffs, mon, cv = pl.pallas_call(
        body,
        grid=(n_blocks,),
        in_specs=[
            pl.BlockSpec((bn, d), lambda i: (i, 0)),
            pl.BlockSpec((h_dim, d), lambda i: (0, 0)),
            pl.BlockSpec((1, h_dim), lambda i: (0, 0)),
            pl.BlockSpec((e_dim, h_dim), lambda i: (0, 0)),
            pl.BlockSpec((1, e_dim), lambda i: (0, 0)),
            pl.BlockSpec((1, e_dim), lambda i: (0, 0)),
        ],
        out_specs=[
            pl.BlockSpec((bn, e_dim), lambda i: (i, 0)),
            pl.BlockSpec((1, 1), lambda i: (0, 0), memory_space=pltpu.SMEM),
            pl.BlockSpec((1, 1), lambda i: (0, 0), memory_space=pltpu.SMEM),
        ],
        out_shape=[
            jax.ShapeDtypeStruct((n, e_dim), jnp.float32),
            jax.ShapeDtypeStruct((1, 1), jnp.float32),
            jax.ShapeDtypeStruct((1, 1), jnp.float32),
        ],
    )(global_features, W1, b1.reshape(1, h_dim), W2,
      b2.reshape(1, e_dim), ema_load.reshape(1, e_dim))
    return coeffs, mon[0, 0], cv[0, 0]


# speedup vs baseline: 6.2757x; 1.1700x over previous
import functools

import jax
import jax.numpy as jnp
from jax.experimental import pallas as pl
from jax.experimental.pallas import tpu as pltpu


_K = 8


def _router_body(x_ref, w1_ref, b1_ref, w2_ref, b2_ref, ema_ref,
                 coeffs_ref, mon_ref, cv_ref, *, n_blocks, n_rows, n_experts):
    i = pl.program_id(0)

    z = jax.lax.dot_general(x_ref[...], w1_ref[...],
                            (((1,), (1,)), ((), ())),
                            preferred_element_type=jnp.float32)
    h = jax.nn.silu(z + b1_ref[...])
    logits = jax.lax.dot_general(h, w2_ref[...],
                                 (((1,), (1,)), ((), ())),
                                 preferred_element_type=jnp.float32)
    scores = jax.nn.sigmoid(logits + b2_ref[...])

    masked = scores
    sel = jnp.zeros(scores.shape, jnp.bool_)
    for _ in range(_K):
        elig = masked == jnp.max(masked, axis=1, keepdims=True)
        sel = jnp.logical_or(sel, elig)
        masked = jnp.where(elig, -1.0, masked)
    count = jnp.sum(sel.astype(jnp.int32), axis=1)
    bad = jnp.any(count != _K)

    @pl.when(bad)
    def _exact_topk():
        iota = jax.lax.broadcasted_iota(jnp.int32, scores.shape, 1)
        masked = scores
        sel = jnp.zeros(scores.shape, jnp.bool_)
        for _ in range(_K):
            m = jnp.max(masked, axis=1, keepdims=True)
            elig = masked == m
            fidx = jnp.min(jnp.where(elig, iota, n_experts), axis=1,
                           keepdims=True)
            first = iota == fidx
            sel = jnp.logical_or(sel, first)
            masked = jnp.where(first, -1.0, masked)
        selscores = jnp.where(sel, scores, 0.0)
        denom = jnp.sum(selscores, axis=1, keepdims=True) + 1e-8
        coeffs_ref[...] = selscores / denom

    @pl.when(jnp.logical_not(bad))
    def _fast_topk():
        selscores = jnp.where(sel, scores, 0.0)
        denom = jnp.sum(selscores, axis=1, keepdims=True) + 1e-8
        coeffs_ref[...] = selscores / denom

    part = jnp.sum(jnp.max(coeffs_ref[...], axis=1))

    @pl.when(i == 0)
    def _init():
        mon_ref[0, 0] = 0.0
        e = ema_ref[...]
        mu = jnp.sum(e) / n_experts
        var = jnp.sum((e - mu) ** 2) / (n_experts - 1)
        cv_ref[0, 0] = jnp.sqrt(var) / (mu + 1e-8)

    mon_ref[0, 0] = mon_ref[0, 0] + part

    @pl.when(i == n_blocks - 1)
    def _final():
        mon_ref[0, 0] = mon_ref[0, 0] / n_rows


def kernel(global_features, W1, b1, W2, b2, ema_load):
    n, d = global_features.shape
    h_dim = W1.shape[0]
    e_dim = W2.shape[0]
    bn = 512
    n_blocks = n // bn

    body = functools.partial(_router_body, n_blocks=n_blocks, n_rows=n,
                             n_experts=e_dim)
    coeffs, mon, cv = pl.pallas_call(
        body,
        grid=(n_blocks,),
        in_specs=[
            pl.BlockSpec((bn, d), lambda i: (i, 0)),
            pl.BlockSpec((h_dim, d), lambda i: (0, 0)),
            pl.BlockSpec((1, h_dim), lambda i: (0, 0)),
            pl.BlockSpec((e_dim, h_dim), lambda i: (0, 0)),
            pl.BlockSpec((1, e_dim), lambda i: (0, 0)),
            pl.BlockSpec((1, e_dim), lambda i: (0, 0)),
        ],
        out_specs=[
            pl.BlockSpec((bn, e_dim), lambda i: (i, 0)),
            pl.BlockSpec((1, 1), lambda i: (0, 0), memory_space=pltpu.SMEM),
            pl.BlockSpec((1, 1), lambda i: (0, 0), memory_space=pltpu.SMEM),
        ],
        out_shape=[
            jax.ShapeDtypeStruct((n, e_dim), jnp.float32),
            jax.ShapeDtypeStruct((1, 1), jnp.float32),
            jax.ShapeDtypeStruct((1, 1), jnp.float32),
        ],
    )(global_features, W1, b1.reshape(1, h_dim), W2,
      b2.reshape(1, e_dim), ema_load.reshape(1, e_dim))
    return coeffs, mon[0, 0], cv[0, 0]
